# Initial kernel scaffold; baseline (speedup 1.0000x reference)
import jax
import jax.numpy as jnp
from jax.experimental import pallas as pl


def kernel(X, edge_index, A_values, W, b):
    raise NotImplementedError("write your pallas kernel here")


# speedup vs baseline: 2.9983x; 2.9983x over previous
import functools

import jax
import jax.numpy as jnp
from jax import lax
from jax.experimental import pallas as pl
from jax.experimental.pallas import tpu as pltpu
from jax.experimental.pallas import tpu_sc as plsc

N = 10000
N_PAD = 10240
E = 320000
F = 128
FH = F // 2

NC = 2
NS = 16
NW = NC * NS

C = 80
CHUNKS_PER_TILE = E // (NS * C)
ROWS_PER_TILE = N_PAD // NS
ZROWS = 128

MM_BLOCK = 400


def _matmul_body(x_ref, w_ref, o_ref):
    xw = jnp.dot(x_ref[...], w_ref[...], preferred_element_type=jnp.float32)
    o_ref[0] = xw[:, :FH]
    o_ref[1] = xw[:, FH:]


def _matmul_split(X, W):
    return pl.pallas_call(
        _matmul_body,
        grid=(N // MM_BLOCK,),
        in_specs=[
            pl.BlockSpec((MM_BLOCK, F), lambda i: (i, 0)),
            pl.BlockSpec((F, F), lambda i: (0, 0)),
        ],
        out_specs=pl.BlockSpec((NC, MM_BLOCK, FH), lambda i: (0, i, 0)),
        out_shape=jax.ShapeDtypeStruct((NC, N, FH), jnp.float32),
    )(X, W)


def _combine_body(p_ref, b_ref, o_ref):
    o_ref[:, :FH] = p_ref[0] + b_ref[:, :FH]
    o_ref[:, FH:] = p_ref[1] + b_ref[:, FH:]


def _combine(partials, b2d):
    return pl.pallas_call(
        _combine_body,
        grid=(N // MM_BLOCK,),
        in_specs=[
            pl.BlockSpec((NC, MM_BLOCK, FH), lambda i: (0, i, 0)),
            pl.BlockSpec((1, F), lambda i: (0, 0)),
        ],
        out_specs=pl.BlockSpec((MM_BLOCK, F), lambda i: (i, 0)),
        out_shape=jax.ShapeDtypeStruct((N, F), jnp.float32),
    )(partials, b2d)


def _sc_body(xw_hbm, src_hbm, dst_hbm, vals_hbm, out_hbm,
             src_v, dst_v, vals_v, rows_v, zbuf, acc, sem):
    cid = lax.axis_index("c")
    sid = lax.axis_index("s")

    pltpu.sync_copy(src_hbm.at[sid], src_v)
    pltpu.sync_copy(dst_hbm.at[sid], dst_v)
    pltpu.sync_copy(vals_hbm.at[sid], vals_v)

    zero = jnp.zeros((16,), jnp.float32)

    def zero_row(i, carry):
        for j in range(FH // 16):
            zbuf[i, pl.ds(j * 16, 16)] = zero
        return carry

    lax.fori_loop(0, ZROWS, zero_row, 0)
    for k in range(ROWS_PER_TILE // ZROWS):
        pltpu.sync_copy(zbuf, acc.at[pl.ds(sid * ROWS_PER_TILE + k * ZROWS,
                                           ZROWS)])
    plsc.subcore_barrier()

    def chunk_body(c, carry):
        pltpu.async_copy(xw_hbm.at[cid].at[src_v.at[c]], rows_v, sem).wait()

        def group_body(g, inner):
            vv = vals_v[c, pl.ds(g * 16, 16)]
            for l in range(16):
                v = vv[l]
                base = g * 16 + l
                for j in range(FH // 16):
                    sl = pl.ds(j * 16, 16)
                    rows_v[base, sl] = rows_v[base, sl] * v
            return inner

        lax.fori_loop(0, C // 16, group_body, 0)
        pltpu.sync_copy(rows_v, acc.at[dst_v.at[c]], add=True)
        return carry

    lax.fori_loop(0, CHUNKS_PER_TILE, chunk_body, 0)
    plsc.subcore_barrier()

    for k in range(ROWS_PER_TILE // ZROWS):
        r0 = sid * ROWS_PER_TILE + k * ZROWS
        pltpu.sync_copy(acc.at[pl.ds(r0, ZROWS)],
                        out_hbm.at[cid, pl.ds(r0, ZROWS)])


_sc_scatter = functools.partial(
    pl.kernel,
    out_type=jax.ShapeDtypeStruct((NC, N_PAD, FH), jnp.float32),
    mesh=plsc.VectorSubcoreMesh(core_axis_name="c", subcore_axis_name="s"),
    compiler_params=pltpu.CompilerParams(use_tc_tiling_on_sc=False),
    scratch_types=[
        pltpu.VMEM((CHUNKS_PER_TILE, C), jnp.int32),
        pltpu.VMEM((CHUNKS_PER_TILE, C), jnp.int32),
        pltpu.VMEM((CHUNKS_PER_TILE, C), jnp.float32),
        pltpu.VMEM((C, FH), jnp.float32),
        pltpu.VMEM((ZROWS, FH), jnp.float32),
        pltpu.VMEM_SHARED((N_PAD, FH), jnp.float32),
        pltpu.SemaphoreType.DMA,
    ],
)(_sc_body)


def kernel(X, edge_index, A_values, W, b):
    XWs = _matmul_split(X, W)
    shape3 = (NS, CHUNKS_PER_TILE, C)
    dst = edge_index[0].astype(jnp.int32).reshape(shape3)
    src = edge_index[1].astype(jnp.int32).reshape(shape3)
    vals = A_values.reshape(shape3)
    partials = _sc_scatter(XWs, src, dst, vals)
    return _combine(partials, b.reshape(1, F))


# speedup vs baseline: 8.7372x; 2.9141x over previous
import functools

import jax
import jax.numpy as jnp
from jax import lax
from jax.experimental import pallas as pl
from jax.experimental.pallas import tpu as pltpu
from jax.experimental.pallas import tpu_sc as plsc

N = 10000
N_PAD = 10240
E = 320000
F = 128
FH = F // 2

NC = 2
NS = 16
NW = NC * NS

C = 80
CHUNKS_PER_TILE = E // (NS * C)
ROWS_PER_TILE = N_PAD // NS
ZROWS = 128

MM_BLOCK = 400


def _matmul_body(x_ref, w_ref, o_ref):
    xw = jnp.dot(x_ref[...], w_ref[...], preferred_element_type=jnp.float32)
    o_ref[0] = xw[:, :FH]
    o_ref[1] = xw[:, FH:]


def _matmul_split(X, W):
    return pl.pallas_call(
        _matmul_body,
        grid=(N // MM_BLOCK,),
        in_specs=[
            pl.BlockSpec((MM_BLOCK, F), lambda i: (i, 0)),
            pl.BlockSpec((F, F), lambda i: (0, 0)),
        ],
        out_specs=pl.BlockSpec((NC, MM_BLOCK, FH), lambda i: (0, i, 0)),
        out_shape=jax.ShapeDtypeStruct((NC, N, FH), jnp.float32),
    )(X, W)


def _combine_body(p_ref, b_ref, o_ref):
    o_ref[:, :FH] = p_ref[0] + b_ref[:, :FH]
    o_ref[:, FH:] = p_ref[1] + b_ref[:, FH:]


def _combine(partials, b2d):
    return pl.pallas_call(
        _combine_body,
        grid=(N // MM_BLOCK,),
        in_specs=[
            pl.BlockSpec((NC, MM_BLOCK, FH), lambda i: (0, i, 0)),
            pl.BlockSpec((1, F), lambda i: (0, 0)),
        ],
        out_specs=pl.BlockSpec((MM_BLOCK, F), lambda i: (i, 0)),
        out_shape=jax.ShapeDtypeStruct((N, F), jnp.float32),
    )(partials, b2d)


def _sc_body(xw_hbm, src_hbm, dst_hbm, vals_hbm, out_hbm,
             src_v, dst_v, vals_v, g0, g1, s0, s1, zbuf, acc,
             gsem0, gsem1, ssem0, ssem1):
    cid = lax.axis_index("c")
    sid = lax.axis_index("s")

    pltpu.sync_copy(src_hbm.at[sid], src_v)
    pltpu.sync_copy(dst_hbm.at[sid], dst_v)
    pltpu.sync_copy(vals_hbm.at[sid], vals_v)

    zero = jnp.zeros((16,), jnp.float32)

    def zero_row(i, carry):
        for j in range(FH // 16):
            zbuf[i, pl.ds(j * 16, 16)] = zero
        return carry

    lax.fori_loop(0, ZROWS, zero_row, 0)
    for k in range(ROWS_PER_TILE // ZROWS):
        pltpu.sync_copy(zbuf, acc.at[pl.ds(sid * ROWS_PER_TILE + k * ZROWS,
                                           ZROWS)])
    plsc.subcore_barrier()

    xw = xw_hbm.at[cid]

    def start_gather(c, buf, sem):
        pltpu.async_copy(xw.at[src_v.at[c]], buf, sem)

    def wait_gather(c, buf, sem):
        pltpu.make_async_copy(xw.at[src_v.at[c]], buf, sem).wait()

    def start_scatter(c, buf, sem):
        pltpu.async_copy(buf, acc.at[dst_v.at[c]], sem, add=True)

    def wait_scatter(c, buf, sem):
        pltpu.make_async_copy(buf, acc.at[dst_v.at[c]], sem).wait()

    def scale(c, gbuf, sbuf):
        def group_body(g, inner):
            vv = vals_v[c, pl.ds(g * 16, 16)]
            for l in range(16):
                v = vv[l]
                base = g * 16 + l
                for j in range(FH // 16):
                    sl = pl.ds(j * 16, 16)
                    sbuf[base, sl] = gbuf[base, sl] * v
            return inner

        lax.fori_loop(0, C // 16, group_body, 0)

    CH = CHUNKS_PER_TILE
    start_gather(0, g0, gsem0)
    start_gather(1, g1, gsem1)
    wait_gather(0, g0, gsem0)
    scale(0, g0, s0)
    start_scatter(0, s0, ssem0)
    start_gather(2, g0, gsem0)
    wait_gather(1, g1, gsem1)
    scale(1, g1, s1)
    start_scatter(1, s1, ssem1)

    @pl.loop(2, CH - 2, step=2)
    def _pairs(c):
        start_gather(c + 1, g1, gsem1)
        wait_gather(c, g0, gsem0)
        wait_scatter(c - 2, s0, ssem0)
        scale(c, g0, s0)
        start_scatter(c, s0, ssem0)
        start_gather(c + 2, g0, gsem0)
        wait_gather(c + 1, g1, gsem1)
        wait_scatter(c - 1, s1, ssem1)
        scale(c + 1, g1, s1)
        start_scatter(c + 1, s1, ssem1)

    start_gather(CH - 1, g1, gsem1)
    wait_gather(CH - 2, g0, gsem0)
    wait_scatter(CH - 4, s0, ssem0)
    scale(CH - 2, g0, s0)
    start_scatter(CH - 2, s0, ssem0)
    wait_gather(CH - 1, g1, gsem1)
    wait_scatter(CH - 3, s1, ssem1)
    scale(CH - 1, g1, s1)
    start_scatter(CH - 1, s1, ssem1)
    wait_scatter(CH - 2, s0, ssem0)
    wait_scatter(CH - 1, s1, ssem1)
    plsc.subcore_barrier()

    for k in range(ROWS_PER_TILE // ZROWS):
        r0 = sid * ROWS_PER_TILE + k * ZROWS
        pltpu.sync_copy(acc.at[pl.ds(r0, ZROWS)],
                        out_hbm.at[cid, pl.ds(r0, ZROWS)])


_sc_scatter = functools.partial(
    pl.kernel,
    out_type=jax.ShapeDtypeStruct((NC, N_PAD, FH), jnp.float32),
    mesh=plsc.VectorSubcoreMesh(core_axis_name="c", subcore_axis_name="s"),
    compiler_params=pltpu.CompilerParams(use_tc_tiling_on_sc=False),
    scratch_types=[
        pltpu.VMEM((CHUNKS_PER_TILE, C), jnp.int32),
        pltpu.VMEM((CHUNKS_PER_TILE, C), jnp.int32),
        pltpu.VMEM((CHUNKS_PER_TILE, C), jnp.float32),
        pltpu.VMEM((C, FH), jnp.float32),
        pltpu.VMEM((C, FH), jnp.float32),
        pltpu.VMEM((C, FH), jnp.float32),
        pltpu.VMEM((C, FH), jnp.float32),
        pltpu.VMEM((ZROWS, FH), jnp.float32),
        pltpu.VMEM_SHARED((N_PAD, FH), jnp.float32),
        pltpu.SemaphoreType.DMA,
        pltpu.SemaphoreType.DMA,
        pltpu.SemaphoreType.DMA,
        pltpu.SemaphoreType.DMA,
    ],
)(_sc_body)


def kernel(X, edge_index, A_values, W, b):
    XWs = _matmul_split(X, W)
    shape3 = (NS, CHUNKS_PER_TILE, C)
    dst = edge_index[0].astype(jnp.int32).reshape(shape3)
    src = edge_index[1].astype(jnp.int32).reshape(shape3)
    vals = A_values.reshape(shape3)
    partials = _sc_scatter(XWs, src, dst, vals)
    return _combine(partials, b.reshape(1, F))


# speedup vs baseline: 10.4473x; 1.1957x over previous
import functools

import jax
import jax.numpy as jnp
from jax import lax
from jax.experimental import pallas as pl
from jax.experimental.pallas import tpu as pltpu
from jax.experimental.pallas import tpu_sc as plsc

N = 10000
N_PAD = 10240
E = 320000
F = 128
FH = F // 2

NC = 2
NS = 16
NW = NC * NS

C = 80
CHUNKS_PER_TILE = E // (NS * C)
ROWS_PER_TILE = N_PAD // NS
ZROWS = 128

MM_BLOCK = 2000
CB_BLOCK = 2000


def _matmul_body(x_ref, w_ref, o_ref):
    o_ref[...] = jnp.dot(x_ref[...], w_ref[...],
                         preferred_element_type=jnp.float32)


def _matmul(X, W):
    return pl.pallas_call(
        _matmul_body,
        grid=(N // MM_BLOCK,),
        in_specs=[
            pl.BlockSpec((MM_BLOCK, F), lambda i: (i, 0)),
            pl.BlockSpec((F, F), lambda i: (0, 0)),
        ],
        out_specs=pl.BlockSpec((MM_BLOCK, F), lambda i: (i, 0)),
        out_shape=jax.ShapeDtypeStruct((N, F), jnp.float32),
    )(X, W)


def _combine_body(p_ref, b_ref, o_ref):
    o_ref[:, :FH] = p_ref[0] + b_ref[:, :FH]
    o_ref[:, FH:] = p_ref[1] + b_ref[:, FH:]


def _combine(partials, b2d):
    return pl.pallas_call(
        _combine_body,
        grid=(N // CB_BLOCK,),
        in_specs=[
            pl.BlockSpec((NC, CB_BLOCK, FH), lambda i: (0, i, 0)),
            pl.BlockSpec((1, F), lambda i: (0, 0)),
        ],
        out_specs=pl.BlockSpec((CB_BLOCK, F), lambda i: (i, 0)),
        out_shape=jax.ShapeDtypeStruct((N, F), jnp.float32),
    )(partials, b2d)


def _sc_body(xw_hbm, ei_hbm, vals_hbm, out_hbm,
             src_v, dst_v, vals_v, g0, g1, s0, s1, zbuf, acc,
             gsem0, gsem1, ssem0, ssem1):
    cid = lax.axis_index("c")
    sid = lax.axis_index("s")

    pltpu.sync_copy(ei_hbm.at[0, sid], dst_v)
    pltpu.sync_copy(ei_hbm.at[1, sid], src_v)
    pltpu.sync_copy(vals_hbm.at[sid], vals_v)

    def xform_row(c, carry):
        for g in range(C // 16):
            sl = pl.ds(g * 16, 16)
            src_v[c, sl] = src_v[c, sl] * 2 + cid
        return carry

    lax.fori_loop(0, CHUNKS_PER_TILE, xform_row, 0)

    zero = jnp.zeros((16,), jnp.float32)

    def zero_row(i, carry):
        for j in range(FH // 16):
            zbuf[i, pl.ds(j * 16, 16)] = zero
        return carry

    lax.fori_loop(0, ZROWS, zero_row, 0)
    for k in range(ROWS_PER_TILE // ZROWS):
        pltpu.sync_copy(zbuf, acc.at[pl.ds(sid * ROWS_PER_TILE + k * ZROWS,
                                           ZROWS)])
    plsc.subcore_barrier()

    def start_gather(c, buf, sem):
        pltpu.async_copy(xw_hbm.at[src_v.at[c]], buf, sem)

    def wait_gather(c, buf, sem):
        pltpu.make_async_copy(xw_hbm.at[src_v.at[c]], buf, sem).wait()

    def start_scatter(c, buf, sem):
        pltpu.async_copy(buf, acc.at[dst_v.at[c]], sem, add=True)

    def wait_scatter(c, buf, sem):
        pltpu.make_async_copy(buf, acc.at[dst_v.at[c]], sem).wait()

    def scale(c, gbuf, sbuf):
        for g in range(C // 16):
            vv = vals_v[c, pl.ds(g * 16, 16)]
            for l in range(16):
                v = vv[l]
                base = g * 16 + l
                for j in range(FH // 16):
                    sl = pl.ds(j * 16, 16)
                    sbuf[base, sl] = gbuf[base, sl] * v

    CH = CHUNKS_PER_TILE
    start_gather(0, g0, gsem0)
    start_gather(1, g1, gsem1)
    wait_gather(0, g0, gsem0)
    scale(0, g0, s0)
    start_scatter(0, s0, ssem0)
    start_gather(2, g0, gsem0)
    wait_gather(1, g1, gsem1)
    scale(1, g1, s1)
    start_scatter(1, s1, ssem1)

    @pl.loop(2, CH - 2, step=2)
    def _pairs(c):
        start_gather(c + 1, g1, gsem1)
        wait_gather(c, g0, gsem0)
        wait_scatter(c - 2, s0, ssem0)
        scale(c, g0, s0)
        start_scatter(c, s0, ssem0)
        start_gather(c + 2, g0, gsem0)
        wait_gather(c + 1, g1, gsem1)
        wait_scatter(c - 1, s1, ssem1)
        scale(c + 1, g1, s1)
        start_scatter(c + 1, s1, ssem1)

    start_gather(CH - 1, g1, gsem1)
    wait_gather(CH - 2, g0, gsem0)
    wait_scatter(CH - 4, s0, ssem0)
    scale(CH - 2, g0, s0)
    start_scatter(CH - 2, s0, ssem0)
    wait_gather(CH - 1, g1, gsem1)
    wait_scatter(CH - 3, s1, ssem1)
    scale(CH - 1, g1, s1)
    start_scatter(CH - 1, s1, ssem1)
    wait_scatter(CH - 2, s0, ssem0)
    wait_scatter(CH - 1, s1, ssem1)
    plsc.subcore_barrier()

    for k in range(ROWS_PER_TILE // ZROWS):
        r0 = sid * ROWS_PER_TILE + k * ZROWS
        pltpu.sync_copy(acc.at[pl.ds(r0, ZROWS)],
                        out_hbm.at[cid, pl.ds(r0, ZROWS)])


_sc_scatter = functools.partial(
    pl.kernel,
    out_type=jax.ShapeDtypeStruct((NC, N_PAD, FH), jnp.float32),
    mesh=plsc.VectorSubcoreMesh(core_axis_name="c", subcore_axis_name="s"),
    compiler_params=pltpu.CompilerParams(use_tc_tiling_on_sc=False),
    scratch_types=[
        pltpu.VMEM((CHUNKS_PER_TILE, C), jnp.int32),
        pltpu.VMEM((CHUNKS_PER_TILE, C), jnp.int32),
        pltpu.VMEM((CHUNKS_PER_TILE, C), jnp.float32),
        pltpu.VMEM((C, FH), jnp.float32),
        pltpu.VMEM((C, FH), jnp.float32),
        pltpu.VMEM((C, FH), jnp.float32),
        pltpu.VMEM((C, FH), jnp.float32),
        pltpu.VMEM((ZROWS, FH), jnp.float32),
        pltpu.VMEM_SHARED((N_PAD, FH), jnp.float32),
        pltpu.SemaphoreType.DMA,
        pltpu.SemaphoreType.DMA,
        pltpu.SemaphoreType.DMA,
        pltpu.SemaphoreType.DMA,
    ],
)(_sc_body)


def kernel(X, edge_index, A_values, W, b):
    XW = _matmul(X, W)
    xw2 = XW.reshape(2 * N, FH)
    ei4 = edge_index.astype(jnp.int32).reshape(2, NS, CHUNKS_PER_TILE, C)
    vals3 = A_values.reshape(NS, CHUNKS_PER_TILE, C)
    partials = _sc_scatter(xw2, ei4, vals3)
    return _combine(partials, b.reshape(1, F))
